# baseline (speedup 1.0000x reference)
import functools

import jax
import jax.numpy as jnp
from jax import lax
from jax.experimental import pallas as pl
from jax.experimental.pallas import tpu as pltpu
from jax.experimental.pallas import tpu_sc as plsc

_INFO = plsc.get_sparse_core_info()
_NC, _NS, _L = _INFO.num_cores, _INFO.num_subcores, _INFO.num_lanes
_NW = _NC * _NS


def kernel(hash_ids, tables):
    batch, n_hb = hash_ids.shape
    n_hb2, vocab, d = tables.shape
    assert n_hb == n_hb2
    rows_total = batch * n_hb
    assert rows_total % _NW == 0
    rpw = rows_total // _NW
    assert rpw % _L == 0

    flat_tables = tables.reshape(n_hb * vocab, d)
    ids_flat = hash_ids.reshape(rows_total).astype(jnp.int32)

    mesh = plsc.VectorSubcoreMesh(core_axis_name="c", subcore_axis_name="s")

    @functools.partial(
        pl.kernel,
        mesh=mesh,
        out_type=jax.ShapeDtypeStruct((rows_total, d), jnp.float32),
        compiler_params=pltpu.CompilerParams(use_tc_tiling_on_sc=False),
        scratch_types=[
            pltpu.VMEM((rpw,), jnp.int32),
            pltpu.VMEM((rpw, d), jnp.float32),
            pltpu.SemaphoreType.DMA,
        ],
    )
    def sc_gather(ids_hbm, tab_hbm, out_hbm, idx_v, rows_v, sem):
        wid = lax.axis_index("s") * _NC + lax.axis_index("c")
        base = wid * rpw
        pltpu.sync_copy(ids_hbm.at[pl.ds(base, rpw)], idx_v)

        lane = lax.iota(jnp.int32, _L)

        def add_offsets(j, carry):
            r = base + j * _L + lane
            off = lax.rem(r, n_hb) * vocab
            sl = pl.ds(j * _L, _L)
            idx_v[sl] = idx_v[sl] + off
            return carry

        lax.fori_loop(0, rpw // _L, add_offsets, 0)

        pltpu.async_copy(tab_hbm.at[idx_v], rows_v, sem).wait()
        pltpu.sync_copy(rows_v, out_hbm.at[pl.ds(base, rpw)])

    out = sc_gather(ids_flat, flat_tables)
    return out.reshape(batch, n_hb, d)


# speedup vs baseline: 5.9255x; 5.9255x over previous
import functools

import jax
import jax.numpy as jnp
from jax import lax
from jax.experimental import pallas as pl
from jax.experimental.pallas import tpu as pltpu
from jax.experimental.pallas import tpu_sc as plsc

_INFO = plsc.get_sparse_core_info()
_NC, _NS, _L = _INFO.num_cores, _INFO.num_subcores, _INFO.num_lanes
_NW = _NC * _NS


def kernel(hash_ids, tables):
    batch, n_hb = hash_ids.shape
    n_hb2, vocab, d = tables.shape
    assert n_hb == n_hb2
    assert d == _NW, "one subcore per embed lane"
    assert batch % _L == 0

    tabs_t = jnp.transpose(tables, (0, 2, 1))
    ids_flat = hash_ids.T.reshape(batch * n_hb).astype(jnp.int32)

    mesh = plsc.VectorSubcoreMesh(core_axis_name="c", subcore_axis_name="s")

    @functools.partial(
        pl.kernel,
        mesh=mesh,
        out_type=jax.ShapeDtypeStruct((n_hb, d, batch), jnp.float32),
        compiler_params=pltpu.CompilerParams(
            use_tc_tiling_on_sc=True, needs_layout_passes=False
        ),
        scratch_types=[
            pltpu.VMEM((batch,), jnp.int32),
            pltpu.VMEM((vocab,), jnp.float32),
            pltpu.VMEM((batch,), jnp.float32),
        ],
    )
    def sc_embed(ids_hbm, tab_hbm, out_hbm, ids_v, row_v, out_v):
        e = lax.axis_index("s") * _NC + lax.axis_index("c")

        def per_block(i, carry):
            pltpu.sync_copy(ids_hbm.at[pl.ds(i * batch, batch)], ids_v)
            pltpu.sync_copy(tab_hbm.at[i, e, :], row_v)

            def gather16(k, c):
                sl = pl.ds(k * _L, _L)
                out_v[sl] = plsc.load_gather(row_v, [ids_v[sl]])
                return c

            lax.fori_loop(0, batch // _L, gather16, 0)
            pltpu.sync_copy(out_v, out_hbm.at[i, e, :])
            return carry

        lax.fori_loop(0, n_hb, per_block, 0)

    out = sc_embed(ids_flat, tabs_t)
    return jnp.transpose(out, (2, 0, 1))


# speedup vs baseline: 6.4929x; 1.0958x over previous
import functools

import jax
import jax.numpy as jnp
from jax import lax
from jax.experimental import pallas as pl
from jax.experimental.pallas import tpu as pltpu
from jax.experimental.pallas import tpu_sc as plsc

_INFO = plsc.get_sparse_core_info()
_NC, _NS, _L = _INFO.num_cores, _INFO.num_subcores, _INFO.num_lanes
_NW = _NC * _NS


def kernel(hash_ids, tables):
    batch, n_hb = hash_ids.shape
    n_hb2, vocab, d = tables.shape
    assert n_hb == n_hb2
    assert d == _NW, "one subcore per embed lane"
    assert batch % _L == 0

    tabs_t = jnp.transpose(tables, (0, 2, 1))
    ids_flat = hash_ids.T.reshape(batch * n_hb).astype(jnp.int32)

    mesh = plsc.VectorSubcoreMesh(core_axis_name="c", subcore_axis_name="s")

    @functools.partial(
        pl.kernel,
        mesh=mesh,
        out_type=jax.ShapeDtypeStruct((n_hb, d, batch), jnp.float32),
        compiler_params=pltpu.CompilerParams(
            use_tc_tiling_on_sc=True, needs_layout_passes=False
        ),
        scratch_types=[
            pltpu.VMEM((vocab,), jnp.float32),
            pltpu.VMEM((batch,), jnp.int32),
            pltpu.VMEM((batch,), jnp.int32),
            pltpu.VMEM((batch,), jnp.float32),
            pltpu.VMEM((batch,), jnp.float32),
            pltpu.SemaphoreType.DMA,
            pltpu.SemaphoreType.DMA,
            pltpu.SemaphoreType.DMA,
            pltpu.SemaphoreType.DMA,
            pltpu.SemaphoreType.DMA,
        ],
    )
    def sc_embed(ids_hbm, tab_hbm, out_hbm, row_v, ids_a, ids_b2, out_a,
                 out_b2, s_row, s_ids_a, s_ids_b, s_out_a, s_out_b):
        e = lax.axis_index("s") * _NC + lax.axis_index("c")
        ids_bufs = (ids_a, ids_b2)
        out_bufs = (out_a, out_b2)
        ids_sems = (s_ids_a, s_ids_b)
        out_sems = (s_out_a, s_out_b)

        def start_row(i):
            return pltpu.async_copy(tab_hbm.at[i, e, :], row_v, s_row)

        def start_ids(i):
            return pltpu.async_copy(
                ids_hbm.at[pl.ds(i * batch, batch)], ids_bufs[i % 2],
                ids_sems[i % 2]
            )

        def gather_block(i):
            ids_b = ids_bufs[i % 2]
            out_b = out_bufs[i % 2]

            @plsc.parallel_loop(0, batch // _L, unroll=8)
            def g16(k):
                sl = pl.ds(k * _L, _L)
                out_b[sl] = plsc.load_gather(row_v, [ids_b[sl]])

        pending_ids = [start_ids(0), start_ids(1)]
        pending_row = [start_row(0)]
        pending_out = []
        for i in range(n_hb):
            pending_ids.pop(0).wait()
            pending_row.pop(0).wait()
            if len(pending_out) > 1:
                pending_out.pop(0).wait()
            gather_block(i)
            if i + 1 < n_hb:
                pending_row.append(start_row(i + 1))
            if i + 2 < n_hb:
                pending_ids.append(start_ids(i + 2))
            pending_out.append(
                pltpu.async_copy(
                    out_bufs[i % 2], out_hbm.at[i, e, :], out_sems[i % 2]
                )
            )
        for p in pending_out:
            p.wait()

    out = sc_embed(ids_flat, tabs_t)
    return jnp.transpose(out, (2, 0, 1))
